# baseline (speedup 1.0000x reference)
import functools

import jax
import jax.numpy as jnp
from jax import lax
from jax.experimental import pallas as pl
from jax.experimental.pallas import tpu as pltpu
from jax.experimental.pallas import tpu_sc as plsc

NC = 2
NS = 16
CH = 128
LAG = 8


def _tc_build_idx(t, B, C, K, blk):
    def body(t_ref, o_ref):
        row = pl.program_id(0) * blk + lax.broadcasted_iota(
            jnp.int32, (blk, K), 0)
        o_ref[...] = t_ref[...] + row * C

    return pl.pallas_call(
        body,
        grid=(B // blk,),
        in_specs=[pl.BlockSpec((blk, K), lambda i: (i, 0))],
        out_specs=pl.BlockSpec((blk, K), lambda i: (i, 0)),
        out_shape=jax.ShapeDtypeStruct((B, K), jnp.int32),
    )(t)


def _sc_gather(x_flat, idx, B, C, K):
    NW = NC * NS
    total = B * K
    per_w = total // NW
    n_ch = per_w // CH
    idx2 = idx.reshape(NW * n_ch, CH)

    mesh = plsc.VectorSubcoreMesh(core_axis_name="c", subcore_axis_name="s")

    @functools.partial(
        pl.kernel,
        out_type=jax.ShapeDtypeStruct((total, 1), jnp.float32),
        mesh=mesh,
        scratch_types=[
            pltpu.VMEM((n_ch, CH), jnp.int32),
            pltpu.VMEM((per_w, 1), jnp.float32),
            pltpu.SemaphoreType.DMA,
        ],
        compiler_params=pltpu.CompilerParams(use_tc_tiling_on_sc=False),
    )
    def gather_kernel(x_hbm, idx_hbm, out_hbm, idx_v, g_v, sem):
        wid = lax.axis_index("s") * NC + lax.axis_index("c")
        base = wid * per_w
        pltpu.sync_copy(idx_hbm.at[pl.ds(wid * n_ch, n_ch), :], idx_v)

        def fire(c, carry):
            pltpu.async_copy(x_hbm.at[idx_v.at[c]],
                             g_v.at[pl.ds(c * CH, CH)], sem).wait()
            return carry

        lax.fori_loop(0, n_ch, fire, 0, unroll=False)

        pltpu.sync_copy(g_v, out_hbm.at[pl.ds(base, per_w)])

    return gather_kernel(x_flat, idx2)


def _tc_loss(x, t, g, B, C, K, blk):
    grid = B // blk

    def body(x_ref, t_ref, g_ref, o_ref):
        e = jnp.exp(x_ref[...])
        q = jnp.sum(e, axis=1)

        tt = t_ref[...]
        dup = jnp.zeros((blk, K), jnp.int32)
        for s in range(1, K):
            eq = (tt[:, s:] == tt[:, : K - s]).astype(jnp.int32)
            dup = dup | jnp.concatenate(
                [jnp.zeros((blk, s), jnp.int32), eq], axis=1)

        eg = jnp.where(dup > 0, 0.0, jnp.exp(g_ref[...]))
        p = jnp.sum(eg, axis=1)

        part = jnp.sum(jnp.log(q) - jnp.log(p)) * (1.0 / B)

        @pl.when(pl.program_id(0) == 0)
        def _():
            o_ref[0, 0] = 0.0

        o_ref[0, 0] += part

    return pl.pallas_call(
        body,
        grid=(grid,),
        in_specs=[
            pl.BlockSpec((blk, C), lambda i: (i, 0)),
            pl.BlockSpec((blk, K), lambda i: (i, 0)),
            pl.BlockSpec((blk, K), lambda i: (i, 0)),
        ],
        out_specs=pl.BlockSpec(memory_space=pltpu.SMEM),
        out_shape=jax.ShapeDtypeStruct((1, 1), jnp.float32),
    )(x, t, g)


@jax.jit
def kernel(input, target):
    B, C = input.shape
    K = target.shape[1]
    x = input.astype(jnp.float32)
    t = target.astype(jnp.int32)

    idx = _tc_build_idx(t, B, C, K, blk=1024)
    g_flat = _sc_gather(x.reshape(B * C, 1), idx, B, C, K)
    g = g_flat.reshape(B, K)

    out = _tc_loss(x, t, g, B, C, K, blk=512)
    return out[0, 0]


# speedup vs baseline: 85.7133x; 85.7133x over previous
import functools

import jax
import jax.numpy as jnp
from jax import lax
from jax.experimental import pallas as pl
from jax.experimental.pallas import tpu as pltpu
from jax.experimental.pallas import tpu_sc as plsc

NC = 2
NS = 16
RC = 32


def _tc_build_w(tT, B, K, blk):
    def body(t_ref, w_ref):
        tt = t_ref[...]
        dup = jnp.zeros((K, blk), jnp.int32)
        for s in range(1, K):
            eq = (tt[s:, :] == tt[: K - s, :]).astype(jnp.int32)
            dup = dup | jnp.concatenate(
                [jnp.zeros((s, blk), jnp.int32), eq], axis=0)
        w_ref[...] = (1 - dup).astype(jnp.float32)

    return pl.pallas_call(
        body,
        grid=(B // blk,),
        in_specs=[pl.BlockSpec((K, blk), lambda i: (0, i))],
        out_specs=pl.BlockSpec((K, blk), lambda i: (0, i)),
        out_shape=jax.ShapeDtypeStruct((K, B), jnp.float32),
    )(tT)


def _sc_gather(x, t_flat, rowpat, B, C, K):
    NW = NC * NS
    RPT = B // NW
    n_ch = RPT // RC
    per_w = RPT * K
    GPC = RC * K

    mesh = plsc.VectorSubcoreMesh(core_axis_name="c", subcore_axis_name="s",
                                  num_cores=NC, num_subcores=NS)

    @functools.partial(
        pl.kernel,
        out_type=jax.ShapeDtypeStruct((B * K,), jnp.float32),
        mesh=mesh,
        scratch_types=[
            pltpu.VMEM((RC, C), jnp.float32),
            pltpu.VMEM((RC, C), jnp.float32),
            pltpu.VMEM((GPC,), jnp.int32),
            pltpu.VMEM((per_w,), jnp.int32),
            pltpu.VMEM((per_w,), jnp.float32),
            pltpu.SemaphoreType.DMA,
            pltpu.SemaphoreType.DMA,
        ],
        compiler_params=pltpu.CompilerParams(needs_layout_passes=False),
    )
    def gather_kernel(x_hbm, t_hbm, rp_hbm, out_hbm, xa, xb, rp_v, t_v, g_v,
                      sem_a, sem_b):
        wid = lax.axis_index("s") * NC + lax.axis_index("c")
        rbase = wid * RPT
        pltpu.sync_copy(rp_hbm, rp_v)
        pltpu.sync_copy(t_hbm.at[pl.ds(wid * per_w, per_w)], t_v)

        bufs = [(xa, sem_a), (xb, sem_b)]

        def fire(k):
            buf, sem = bufs[k % 2]
            return pltpu.async_copy(
                x_hbm.at[pl.ds(rbase + k * RC, RC), :], buf, sem)

        cps = {0: fire(0)}
        for k in range(n_ch):
            if k + 1 < n_ch:
                cps[k + 1] = fire(k + 1)
            cps[k].wait()
            buf, _ = bufs[k % 2]
            for v in range(GPC // 16):
                off = k * GPC + v * 16
                rp = rp_v[pl.ds(v * 16, 16)]
                cols = t_v[pl.ds(off, 16)]
                g_v[pl.ds(off, 16)] = plsc.load_gather(buf, [rp, cols])

        pltpu.sync_copy(g_v, out_hbm.at[pl.ds(wid * per_w, per_w)])

    return gather_kernel(x, t_flat, rowpat)


def _tc_dense_q(x, B, C, blk):
    def body(x_ref, o_ref):
        q = jnp.sum(jnp.exp(x_ref[...]), axis=1)
        part = jnp.sum(jnp.log(q))

        @pl.when(pl.program_id(0) == 0)
        def _():
            o_ref[0, 0] = 0.0

        o_ref[0, 0] += part

    return pl.pallas_call(
        body,
        grid=(B // blk,),
        in_specs=[pl.BlockSpec((blk, C), lambda i: (i, 0))],
        out_specs=pl.BlockSpec(memory_space=pltpu.SMEM),
        out_shape=jax.ShapeDtypeStruct((1, 1), jnp.float32),
    )(x)


def _tc_combine(gT, wT, B, K, blk):
    def body(g_ref, w_ref, o_ref):
        p = jnp.sum(jnp.exp(g_ref[...]) * w_ref[...], axis=0)
        part = jnp.sum(jnp.log(p))

        @pl.when(pl.program_id(0) == 0)
        def _():
            o_ref[0, 0] = 0.0

        o_ref[0, 0] += part

    return pl.pallas_call(
        body,
        grid=(B // blk,),
        in_specs=[pl.BlockSpec((K, blk), lambda i: (0, i)),
                  pl.BlockSpec((K, blk), lambda i: (0, i))],
        out_specs=pl.BlockSpec(memory_space=pltpu.SMEM),
        out_shape=jax.ShapeDtypeStruct((1, 1), jnp.float32),
    )(gT, wT)


@jax.jit
def kernel(input, target):
    B, C = input.shape
    K = target.shape[1]
    x = input.astype(jnp.float32)
    t = target.astype(jnp.int32)
    tT = t.T

    wT = _tc_build_w(tT, B, K, blk=2048)
    rowpat = jnp.repeat(jnp.arange(RC, dtype=jnp.int32), K)
    g_flat = _sc_gather(x, t.reshape(B * K), rowpat, B, C, K)
    gT = g_flat.reshape(B, K).T

    sq = _tc_dense_q(x, B, C, blk=1024)
    sp = _tc_combine(gT, wT, B, K, blk=2048)
    return (sq[0, 0] - sp[0, 0]) * (1.0 / B)
